# baseline (speedup 1.0000x reference)
import functools

import jax
import jax.numpy as jnp
from jax import lax
from jax.experimental import pallas as pl
from jax.experimental.pallas import tpu as pltpu
from jax.experimental.pallas import tpu_sc as plsc

_B, _C, _H, _W = 16, 512, 32, 32
_D = 256


def _sc_body(col_hbm, row_hbm, out_hbm, tile_v, rowbuf, sem):
    nc = 2
    wid = lax.axis_index("s") * nc + lax.axis_index("c")
    pltpu.sync_copy(col_hbm.at[pl.ds(0, _W)], tile_v.at[0, :, pl.ds(0, _D)])
    pltpu.sync_copy(row_hbm.at[pl.ds(wid, 1)], rowbuf)
    for j in range(_D // 16):
        v = rowbuf[0, pl.ds(16 * j, 16)]
        for w in range(_W):
            tile_v[0, w, pl.ds(_D + 16 * j, 16)] = v
    copies = [
        pltpu.async_copy(tile_v, out_hbm.at[b, pl.ds(wid, 1)], sem)
        for b in range(_B)
    ]
    for c in copies:
        c.wait()


def kernel(x, row_embed, col_embed):
    b = x.shape[0]
    mesh = plsc.VectorSubcoreMesh(core_axis_name="c", subcore_axis_name="s")
    sc_fn = functools.partial(
        pl.kernel,
        mesh=mesh,
        out_type=jax.ShapeDtypeStruct((b, _H, _W, _C), jnp.float32),
        scratch_types=[
            pltpu.VMEM((1, _W, _C), jnp.float32),
            pltpu.VMEM((1, _D), jnp.float32),
            pltpu.SemaphoreType.DMA,
        ],
    )(_sc_body)
    out = sc_fn(col_embed, row_embed)
    return jnp.transpose(out, (0, 3, 1, 2))


# speedup vs baseline: 2.6642x; 2.6642x over previous
import jax
import jax.numpy as jnp
from jax.experimental import pallas as pl
from jax.experimental.pallas import tpu as pltpu

_B, _C, _H, _W = 16, 512, 32, 32
_D = 256
_HH = _H // 2


def _pos_kernel(col_ref, row_ref, out_hbm, scratch, sem):
    col = col_ref[0:_W, :]
    row = row_ref[0:_H, :]
    for k in range(2):
        hs = pl.ds(k * _HH, _HH)
        scratch[hs, :, 0:_D] = jnp.broadcast_to(
            col[None, :, :], (_HH, _W, _D))
        scratch[hs, :, _D:_C] = jnp.broadcast_to(
            row[k * _HH:(k + 1) * _HH, None, :], (_HH, _W, _D))
        for b in range(_B):
            pltpu.make_async_copy(
                scratch.at[hs], out_hbm.at[b, hs], sem.at[b]
            ).start()
    for k in range(2):
        hs = pl.ds(k * _HH, _HH)
        for b in range(_B):
            pltpu.make_async_copy(
                scratch.at[hs], out_hbm.at[b, hs], sem.at[b]
            ).wait()


def kernel(x, row_embed, col_embed):
    b = x.shape[0]
    out = pl.pallas_call(
        _pos_kernel,
        in_specs=[
            pl.BlockSpec(memory_space=pltpu.VMEM),
            pl.BlockSpec(memory_space=pltpu.VMEM),
        ],
        out_specs=pl.BlockSpec(memory_space=pl.ANY),
        out_shape=jax.ShapeDtypeStruct((b, _H, _W, _C), jnp.float32),
        scratch_shapes=[
            pltpu.VMEM((_H, _W, _C), jnp.float32),
            pltpu.SemaphoreType.DMA((_B,)),
        ],
    )(col_embed, row_embed)
    return jnp.transpose(out, (0, 3, 1, 2))


# speedup vs baseline: 2.7172x; 1.0199x over previous
import jax
import jax.numpy as jnp
from jax.experimental import pallas as pl
from jax.experimental.pallas import tpu as pltpu

_B, _C, _H, _W = 16, 512, 32, 32
_D = 256


def _pos_kernel(col_ref, row_ref, out_hbm, scratch, sem):
    col = col_ref[0:_W, :]
    row = row_ref[0:_H, :]
    for r in range(2):
        scratch[r, :, :, 0:_D] = jnp.broadcast_to(col[None, :, :], (_H, _W, _D))
        scratch[r, :, :, _D:_C] = jnp.broadcast_to(row[:, None, :], (_H, _W, _D))
    for p in range(_B // 2):
        pltpu.make_async_copy(
            scratch, out_hbm.at[pl.ds(2 * p, 2)], sem.at[p]
        ).start()
    for p in range(_B // 2):
        pltpu.make_async_copy(
            scratch, out_hbm.at[pl.ds(2 * p, 2)], sem.at[p]
        ).wait()


def kernel(x, row_embed, col_embed):
    b = x.shape[0]
    out = pl.pallas_call(
        _pos_kernel,
        in_specs=[
            pl.BlockSpec(memory_space=pltpu.VMEM),
            pl.BlockSpec(memory_space=pltpu.VMEM),
        ],
        out_specs=pl.BlockSpec(memory_space=pl.ANY),
        out_shape=jax.ShapeDtypeStruct((b, _H, _W, _C), jnp.float32),
        scratch_shapes=[
            pltpu.VMEM((2, _H, _W, _C), jnp.float32),
            pltpu.SemaphoreType.DMA((_B // 2,)),
        ],
    )(col_embed, row_embed)
    return jnp.transpose(out, (0, 3, 1, 2))
